# Initial kernel scaffold; baseline (speedup 1.0000x reference)
import jax
import jax.numpy as jnp
from jax.experimental import pallas as pl


def kernel(x, edge_index, W1, b1, beta1, beta2, W2, b2):
    raise NotImplementedError("write your pallas kernel here")


# speedup vs baseline: 4.6793x; 4.6793x over previous
import functools

import jax
import jax.numpy as jnp
from jax import lax
from jax.experimental import pallas as pl
from jax.experimental.pallas import tpu as pltpu
from jax.experimental.pallas import tpu_sc as plsc

N = 50000
D = 100
DP = 112
E = 800000
EPAD = 802816
CH = 16384
NPAD = 65536
ROWS_PER_TILE = CH // 16
B1_EDGES_PER_TILE = EPAD // 32
B2_EDGES_PER_TILE = EPAD // 16
BLK = 128

_mesh = plsc.VectorSubcoreMesh(core_axis_name="c", subcore_axis_name="s")
_sc_params = pltpu.CompilerParams(
    needs_layout_passes=False, use_tc_tiling_on_sc=False)


def _tc1_body(x_ref, w_ref, b_ref, o_ref):
    h = jnp.maximum(
        jnp.dot(x_ref[...], w_ref[...], preferred_element_type=jnp.float32)
        + b_ref[...],
        0.0,
    )
    col = lax.broadcasted_iota(jnp.int32, h.shape, 1)
    hm = jnp.where(col < D, h, 0.0)
    ns = jnp.sqrt(jnp.sum(hm * hm, axis=1, keepdims=True)) + 1e-12
    o_ref[...] = jnp.where(
        col < D, h, jnp.where(col == D, 1.0, jnp.where(col == D + 1, ns, 0.0))
    )


def _tc1(xp, W1p, b1p):
    return pl.pallas_call(
        _tc1_body,
        grid=(50,),
        in_specs=[
            pl.BlockSpec((1000, 64), lambda i: (i, 0)),
            pl.BlockSpec((64, DP), lambda i: (0, 0)),
            pl.BlockSpec((1, DP), lambda i: (0, 0)),
        ],
        out_specs=pl.BlockSpec((1000, DP), lambda i: (i, 0)),
        out_shape=jax.ShapeDtypeStruct((N, DP), jnp.float32),
    )(xp, W1p, b1p)


def _epilogue_rows(acc, hp, beta):
    ns_eps = hp[:, D + 1 : D + 2]
    rs = (ns_eps - 1e-12) ** 2
    w_self = jnp.exp(beta * rs / (ns_eps * ns_eps))
    full = acc + w_self * hp
    denom = full[:, D : D + 1] + 1e-16
    hnext = full / denom
    col = lax.broadcasted_iota(jnp.int32, hnext.shape, 1)
    hm = jnp.where(col < D, hnext, 0.0)
    return hm, col


def _tc2_body(acc_ref, hp_ref, b_ref, o_ref):
    hm, col = _epilogue_rows(acc_ref[...], hp_ref[...], b_ref[0, 0])
    ns = jnp.sqrt(jnp.sum(hm * hm, axis=1, keepdims=True)) + 1e-12
    o_ref[...] = jnp.where(
        col < D, hm, jnp.where(col == D, 1.0, jnp.where(col == D + 1, ns, 0.0))
    )


def _tc2(acc, hp, bscal):
    return pl.pallas_call(
        _tc2_body,
        grid=(50,),
        in_specs=[
            pl.BlockSpec((1000, DP), lambda i: (i, 0)),
            pl.BlockSpec((1000, DP), lambda i: (i, 0)),
            pl.BlockSpec((8, 128), lambda i: (0, 0)),
        ],
        out_specs=pl.BlockSpec((1000, DP), lambda i: (i, 0)),
        out_shape=jax.ShapeDtypeStruct((N, DP), jnp.float32),
    )(acc, hp, bscal)


def _tc3_body(acc_ref, hp_ref, b_ref, w2_ref, b2_ref, o_ref):
    hm, _ = _epilogue_rows(acc_ref[...], hp_ref[...], b_ref[0, 0])
    o_ref[...] = (
        jnp.dot(hm, w2_ref[...], preferred_element_type=jnp.float32)
        + b2_ref[...]
    )


def _tc3(acc, hp, bscal, W2p, b2p):
    return pl.pallas_call(
        _tc3_body,
        grid=(50,),
        in_specs=[
            pl.BlockSpec((1000, DP), lambda i: (i, 0)),
            pl.BlockSpec((1000, DP), lambda i: (i, 0)),
            pl.BlockSpec((8, 128), lambda i: (0, 0)),
            pl.BlockSpec((DP, 128), lambda i: (0, 0)),
            pl.BlockSpec((1, 128), lambda i: (0, 0)),
        ],
        out_specs=pl.BlockSpec((1000, 128), lambda i: (i, 0)),
        out_shape=jax.ShapeDtypeStruct((N, 128), jnp.float32),
    )(acc, hp, bscal, W2p, b2p)


@functools.partial(
    pl.kernel,
    mesh=_mesh,
    out_type=jax.ShapeDtypeStruct((EPAD,), jnp.float32),
    scratch_types=[
        pltpu.VMEM((BLK,), jnp.int32),
        pltpu.VMEM((BLK,), jnp.int32),
        pltpu.VMEM((BLK, DP), jnp.float32),
        pltpu.VMEM((BLK, DP), jnp.float32),
        pltpu.VMEM((BLK,), jnp.float32),
        pltpu.VMEM((16,), jnp.float32),
        pltpu.SemaphoreType.DMA,
        pltpu.SemaphoreType.DMA,
    ],
    compiler_params=_sc_params,
)
def _sc_edge_w(hp_hbm, src_hbm, dst_hbm, bvec_hbm, w_hbm,
               sidx_v, didx_v, srows_v, drows_v, wbuf_v,
               bv_v, sem1, sem2):
    c = lax.axis_index("c")
    s = lax.axis_index("s")
    wid = s * 2 + c
    ebase = wid * B1_EDGES_PER_TILE
    pltpu.sync_copy(bvec_hbm, bv_v)
    bv = bv_v[...]
    babs = jnp.abs(bv)

    lane = lax.iota(jnp.int32, 16)

    def block(g, carry):
        e0 = ebase + g * BLK
        pltpu.sync_copy(src_hbm.at[pl.ds(e0, BLK)], sidx_v)
        pltpu.sync_copy(dst_hbm.at[pl.ds(e0, BLK)], didx_v)
        cp1 = pltpu.async_copy(hp_hbm.at[sidx_v], srows_v, sem1)
        cp2 = pltpu.async_copy(hp_hbm.at[didx_v], drows_v, sem2)
        cp1.wait()
        cp2.wait()

        def group(k, carry2):
            rowidx = k * 16 + lane
            dots = jnp.zeros((16,), jnp.float32)
            for j in range(D):
                cj = jnp.full((16,), j, jnp.int32)
                sv = plsc.load_gather(srows_v, [rowidx, cj])
                dv = plsc.load_gather(drows_v, [rowidx, cj])
                dots = dots + sv * dv
            c101 = jnp.full((16,), D + 1, jnp.int32)
            ns = plsc.load_gather(srows_v, [rowidx, c101])
            nd = plsc.load_gather(drows_v, [rowidx, c101])
            a = bv * dots / (ns * nd)
            a = jnp.minimum(jnp.maximum(a, -babs), babs)
            wbuf_v[pl.ds(k * 16, 16)] = jnp.exp(a)
            return carry2

        lax.fori_loop(0, BLK // 16, group, 0)
        pltpu.sync_copy(wbuf_v, w_hbm.at[pl.ds(e0, BLK)])
        return carry

    lax.fori_loop(0, B1_EDGES_PER_TILE // BLK, block, 0)


@functools.partial(
    pl.kernel,
    mesh=_mesh,
    out_type=jax.ShapeDtypeStruct((NPAD, DP), jnp.float32),
    scratch_types=[
        pltpu.VMEM((BLK,), jnp.int32),
        pltpu.VMEM((BLK,), jnp.int32),
        pltpu.VMEM((BLK,), jnp.float32),
        pltpu.VMEM((BLK,), jnp.int32),
        pltpu.VMEM((BLK, DP), jnp.float32),
        pltpu.VMEM_SHARED((CH, DP), jnp.float32),
        pltpu.SemaphoreType.DMA,
    ],
    compiler_params=_sc_params,
)
def _sc_scatter(hp_hbm, src_hbm, dst_hbm, w_hbm, zrows_hbm, acc_hbm,
                sidx_v, didx_v, wbuf_v, oidx_v, rows_v, shared, sem):
    c = lax.axis_index("c")
    s = lax.axis_index("s")

    for ci in range(2):
        chunk = c * 2 + ci
        nbase = chunk * CH
        pltpu.sync_copy(zrows_hbm, shared.at[pl.ds(s * ROWS_PER_TILE,
                                                   ROWS_PER_TILE)])
        plsc.subcore_barrier()

        def block(g, carry):
            e0 = s * B2_EDGES_PER_TILE + g * BLK
            pltpu.sync_copy(src_hbm.at[pl.ds(e0, BLK)], sidx_v)
            pltpu.sync_copy(dst_hbm.at[pl.ds(e0, BLK)], didx_v)
            pltpu.sync_copy(w_hbm.at[pl.ds(e0, BLK)], wbuf_v)
            for k in range(BLK // 16):
                off = didx_v[pl.ds(k * 16, 16)] - nbase
                m = (off >= 0) & (off < CH)
                oidx_v[pl.ds(k * 16, 16)] = jnp.where(m, off, 0)
                wbuf_v[pl.ds(k * 16, 16)] = jnp.where(
                    m, wbuf_v[pl.ds(k * 16, 16)], 0.0)
            cp = pltpu.async_copy(hp_hbm.at[sidx_v], rows_v, sem)
            cp.wait()

            def scale(k, carry2):
                wv = wbuf_v[pl.ds(k * 16, 16)]
                for j in range(16):
                    e = k * 16 + j
                    w = wv[j]
                    for cth in range(7):
                        rows_v[e, pl.ds(cth * 16, 16)] = (
                            rows_v[e, pl.ds(cth * 16, 16)] * w)
                return carry2

            lax.fori_loop(0, BLK // 16, scale, 0)
            pltpu.sync_copy(rows_v, shared.at[oidx_v], add=True)
            return carry

        lax.fori_loop(0, B2_EDGES_PER_TILE // BLK, block, 0)
        plsc.subcore_barrier()
        pltpu.sync_copy(
            shared.at[pl.ds(s * ROWS_PER_TILE, ROWS_PER_TILE)],
            acc_hbm.at[pl.ds(nbase + s * ROWS_PER_TILE, ROWS_PER_TILE)])
        plsc.subcore_barrier()


def kernel(x, edge_index, W1, b1, beta1, beta2, W2, b2):
    f32 = jnp.float32
    xp = jnp.pad(x, ((0, 0), (0, 64 - x.shape[1])))
    W1p = jnp.pad(W1, ((0, 64 - W1.shape[0]), (0, DP - W1.shape[1])))
    b1p = jnp.pad(b1, (0, DP - b1.shape[0])).at[D].set(1.0).reshape(1, DP)
    W2p = jnp.pad(W2, ((0, DP - W2.shape[0]), (0, 128 - W2.shape[1])))
    b2p = jnp.pad(b2, (0, 128 - b2.shape[0])).reshape(1, 128)

    srcp = jnp.concatenate(
        [edge_index[0], jnp.zeros((EPAD - E,), jnp.int32)])
    dstp = jnp.concatenate(
        [edge_index[1], jnp.full((EPAD - E,), 1 << 20, jnp.int32)])
    zrows = jnp.zeros((ROWS_PER_TILE, DP), f32)

    bvec1 = jnp.full((16,), beta1, f32)
    bvec2 = jnp.full((16,), beta2, f32)
    bscal1 = jnp.full((8, 128), beta1, f32)
    bscal2 = jnp.full((8, 128), beta2, f32)

    hp0 = _tc1(xp, W1p, b1p)
    w1 = _sc_edge_w(hp0, srcp, dstp, bvec1)
    acc1 = _sc_scatter(hp0, srcp, dstp, w1, zrows)
    hp1 = _tc2(acc1, hp0, bscal1)
    w2 = _sc_edge_w(hp1, srcp, dstp, bvec2)
    acc2 = _sc_scatter(hp1, srcp, dstp, w2, zrows)
    out = _tc3(acc2, hp1, bscal2, W2p, b2p)
    return out[:, 0:1]


# speedup vs baseline: 6.0633x; 1.2958x over previous
import functools

import jax
import jax.numpy as jnp
from jax import lax
from jax.experimental import pallas as pl
from jax.experimental.pallas import tpu as pltpu
from jax.experimental.pallas import tpu_sc as plsc

N = 50000
D = 100
DP = 112
E = 800000
EPAD = 802816
CH = 14336
NPAD = 57344
ROWS_PER_TILE = CH // 16
B1_EDGES_PER_TILE = EPAD // 32
B2_EDGES_PER_TILE = EPAD // 16
BLK = 512
SUB = 128
NSUB = BLK // SUB
BLK2 = 256
NSUB2 = BLK2 // SUB

_mesh = plsc.VectorSubcoreMesh(core_axis_name="c", subcore_axis_name="s")
_sc_params = pltpu.CompilerParams(
    needs_layout_passes=False, use_tc_tiling_on_sc=False)


def _tc1_body(x_ref, w_ref, b_ref, o_ref):
    h = jnp.maximum(
        jnp.dot(x_ref[...], w_ref[...], preferred_element_type=jnp.float32)
        + b_ref[...],
        0.0,
    )
    col = lax.broadcasted_iota(jnp.int32, h.shape, 1)
    hm = jnp.where(col < D, h, 0.0)
    ns = jnp.sqrt(jnp.sum(hm * hm, axis=1, keepdims=True)) + 1e-12
    o_ref[...] = jnp.where(
        col < D, h, jnp.where(col == D, 1.0, jnp.where(col == D + 1, ns, 0.0))
    )


def _tc1(xp, W1p, b1p):
    return pl.pallas_call(
        _tc1_body,
        grid=(50,),
        in_specs=[
            pl.BlockSpec((1000, 64), lambda i: (i, 0)),
            pl.BlockSpec((64, DP), lambda i: (0, 0)),
            pl.BlockSpec((1, DP), lambda i: (0, 0)),
        ],
        out_specs=pl.BlockSpec((1000, DP), lambda i: (i, 0)),
        out_shape=jax.ShapeDtypeStruct((N, DP), jnp.float32),
    )(xp, W1p, b1p)


def _epilogue_rows(acc, hp, beta):
    ns_eps = hp[:, D + 1 : D + 2]
    rs = (ns_eps - 1e-12) ** 2
    w_self = jnp.exp(beta * rs / (ns_eps * ns_eps))
    full = acc + w_self * hp
    denom = full[:, D : D + 1] + 1e-16
    hnext = full / denom
    col = lax.broadcasted_iota(jnp.int32, hnext.shape, 1)
    hm = jnp.where(col < D, hnext, 0.0)
    return hm, col


def _tc2_body(acc_ref, hp_ref, b_ref, o_ref):
    hm, col = _epilogue_rows(acc_ref[...], hp_ref[...], b_ref[0, 0])
    ns = jnp.sqrt(jnp.sum(hm * hm, axis=1, keepdims=True)) + 1e-12
    o_ref[...] = jnp.where(
        col < D, hm, jnp.where(col == D, 1.0, jnp.where(col == D + 1, ns, 0.0))
    )


def _tc2(acc, hp, bscal):
    return pl.pallas_call(
        _tc2_body,
        grid=(50,),
        in_specs=[
            pl.BlockSpec((1000, DP), lambda i: (i, 0)),
            pl.BlockSpec((1000, DP), lambda i: (i, 0)),
            pl.BlockSpec((8, 128), lambda i: (0, 0)),
        ],
        out_specs=pl.BlockSpec((1000, DP), lambda i: (i, 0)),
        out_shape=jax.ShapeDtypeStruct((N, DP), jnp.float32),
    )(acc, hp, bscal)


def _tc3_body(acc_ref, hp_ref, b_ref, w2_ref, b2_ref, o_ref):
    hm, _ = _epilogue_rows(acc_ref[...], hp_ref[...], b_ref[0, 0])
    o_ref[...] = (
        jnp.dot(hm, w2_ref[...], preferred_element_type=jnp.float32)
        + b2_ref[...]
    )


def _tc3(acc, hp, bscal, W2p, b2p):
    return pl.pallas_call(
        _tc3_body,
        grid=(50,),
        in_specs=[
            pl.BlockSpec((1000, DP), lambda i: (i, 0)),
            pl.BlockSpec((1000, DP), lambda i: (i, 0)),
            pl.BlockSpec((8, 128), lambda i: (0, 0)),
            pl.BlockSpec((DP, 128), lambda i: (0, 0)),
            pl.BlockSpec((1, 128), lambda i: (0, 0)),
        ],
        out_specs=pl.BlockSpec((1000, 128), lambda i: (i, 0)),
        out_shape=jax.ShapeDtypeStruct((N, 128), jnp.float32),
    )(acc, hp, bscal, W2p, b2p)


@functools.partial(
    pl.kernel,
    mesh=_mesh,
    out_type=jax.ShapeDtypeStruct((EPAD,), jnp.float32),
    scratch_types=[
        pltpu.VMEM((BLK,), jnp.int32),
        pltpu.VMEM((BLK,), jnp.int32),
        pltpu.VMEM((BLK, DP), jnp.float32),
        pltpu.VMEM((BLK, DP), jnp.float32),
        pltpu.VMEM((BLK,), jnp.float32),
        pltpu.VMEM((16,), jnp.float32),
        pltpu.SemaphoreType.DMA,
        pltpu.SemaphoreType.DMA,
    ],
    compiler_params=_sc_params,
)
def _sc_edge_w(hp_hbm, src_hbm, dst_hbm, bvec_hbm, w_hbm,
               sidx_v, didx_v, srows_v, drows_v, wbuf_v,
               bv_v, sem1, sem2):
    c = lax.axis_index("c")
    s = lax.axis_index("s")
    wid = s * 2 + c
    ebase = wid * B1_EDGES_PER_TILE
    pltpu.sync_copy(bvec_hbm, bv_v)
    bv = bv_v[...]
    babs = jnp.abs(bv)

    lane = lax.iota(jnp.int32, 16)

    def block(g, carry):
        e0 = ebase + g * BLK
        ci1 = pltpu.async_copy(src_hbm.at[pl.ds(e0, BLK)], sidx_v, sem1)
        ci2 = pltpu.async_copy(dst_hbm.at[pl.ds(e0, BLK)], didx_v, sem1)
        ci1.wait()
        ci2.wait()
        cps = []
        for j in range(NSUB):
            sl = pl.ds(j * SUB, SUB)
            cps.append(pltpu.async_copy(
                hp_hbm.at[sidx_v.at[sl]], srows_v.at[sl], sem2))
            cps.append(pltpu.async_copy(
                hp_hbm.at[didx_v.at[sl]], drows_v.at[sl], sem2))
        for cp in cps:
            cp.wait()

        def group(k, carry2):
            rowidx = k * 16 + lane
            dots = jnp.zeros((16,), jnp.float32)
            for j in range(D):
                cj = jnp.full((16,), j, jnp.int32)
                sv = plsc.load_gather(srows_v, [rowidx, cj])
                dv = plsc.load_gather(drows_v, [rowidx, cj])
                dots = dots + sv * dv
            c101 = jnp.full((16,), D + 1, jnp.int32)
            ns = plsc.load_gather(srows_v, [rowidx, c101])
            nd = plsc.load_gather(drows_v, [rowidx, c101])
            a = bv * dots / (ns * nd)
            a = jnp.minimum(jnp.maximum(a, -babs), babs)
            wbuf_v[pl.ds(k * 16, 16)] = jnp.exp(a)
            return carry2

        lax.fori_loop(0, BLK // 16, group, 0)
        pltpu.sync_copy(wbuf_v, w_hbm.at[pl.ds(e0, BLK)])
        return carry

    lax.fori_loop(0, B1_EDGES_PER_TILE // BLK, block, 0)


@functools.partial(
    pl.kernel,
    mesh=_mesh,
    out_type=jax.ShapeDtypeStruct((NPAD, DP), jnp.float32),
    scratch_types=[
        pltpu.VMEM((BLK2,), jnp.int32),
        pltpu.VMEM((BLK2,), jnp.int32),
        pltpu.VMEM((BLK2,), jnp.float32),
        pltpu.VMEM((NSUB2, SUB), jnp.int32),
        pltpu.VMEM((BLK2, DP), jnp.float32),
        pltpu.VMEM_SHARED((CH, DP), jnp.float32),
        pltpu.SemaphoreType.DMA,
        pltpu.SemaphoreType.DMA,
    ],
    compiler_params=_sc_params,
)
def _sc_scatter(hp_hbm, src_hbm, dst_hbm, w_hbm, zrows_hbm, acc_hbm,
                sidx_v, didx_v, wbuf_v, oidx_v, rows_v, shared, sem, sem2):
    c = lax.axis_index("c")
    s = lax.axis_index("s")

    for ci in range(2):
        chunk = c * 2 + ci
        nbase = chunk * CH
        pltpu.sync_copy(zrows_hbm, shared.at[pl.ds(s * ROWS_PER_TILE,
                                                   ROWS_PER_TILE)])
        plsc.subcore_barrier()

        def block(g, carry):
            e0 = s * B2_EDGES_PER_TILE + g * BLK2
            ci1 = pltpu.async_copy(src_hbm.at[pl.ds(e0, BLK2)], sidx_v, sem)
            ci2 = pltpu.async_copy(dst_hbm.at[pl.ds(e0, BLK2)], didx_v, sem)
            ci3 = pltpu.async_copy(w_hbm.at[pl.ds(e0, BLK2)], wbuf_v, sem)
            ci1.wait()
            ci2.wait()
            ci3.wait()
            for k in range(BLK2 // 16):
                off = didx_v[pl.ds(k * 16, 16)] - nbase
                m = (off >= 0) & (off < CH)
                oidx_v[k // 8, pl.ds((k % 8) * 16, 16)] = jnp.where(m, off, 0)
                wbuf_v[pl.ds(k * 16, 16)] = jnp.where(
                    m, wbuf_v[pl.ds(k * 16, 16)], 0.0)
            cps = []
            for j in range(NSUB2):
                sl = pl.ds(j * SUB, SUB)
                cps.append(pltpu.async_copy(
                    hp_hbm.at[sidx_v.at[sl]], rows_v.at[sl], sem2))
            for cp in cps:
                cp.wait()

            def scale(k, carry2):
                wv = wbuf_v[pl.ds(k * 16, 16)]
                for j in range(16):
                    e = k * 16 + j
                    w = wv[j]
                    for cth in range(7):
                        rows_v[e, pl.ds(cth * 16, 16)] = (
                            rows_v[e, pl.ds(cth * 16, 16)] * w)
                return carry2

            lax.fori_loop(0, BLK2 // 16, scale, 0)
            for j in range(NSUB2):
                pltpu.sync_copy(rows_v.at[pl.ds(j * SUB, SUB)],
                                shared.at[oidx_v.at[j]], add=True)
            return carry

        lax.fori_loop(0, B2_EDGES_PER_TILE // BLK2, block, 0)
        plsc.subcore_barrier()
        pltpu.sync_copy(
            shared.at[pl.ds(s * ROWS_PER_TILE, ROWS_PER_TILE)],
            acc_hbm.at[pl.ds(nbase + s * ROWS_PER_TILE, ROWS_PER_TILE)])
        plsc.subcore_barrier()


def kernel(x, edge_index, W1, b1, beta1, beta2, W2, b2):
    f32 = jnp.float32
    xp = jnp.pad(x, ((0, 0), (0, 64 - x.shape[1])))
    W1p = jnp.pad(W1, ((0, 64 - W1.shape[0]), (0, DP - W1.shape[1])))
    b1p = jnp.pad(b1, (0, DP - b1.shape[0])).at[D].set(1.0).reshape(1, DP)
    W2p = jnp.pad(W2, ((0, DP - W2.shape[0]), (0, 128 - W2.shape[1])))
    b2p = jnp.pad(b2, (0, 128 - b2.shape[0])).reshape(1, 128)

    srcp = jnp.concatenate(
        [edge_index[0], jnp.zeros((EPAD - E,), jnp.int32)])
    dstp = jnp.concatenate(
        [edge_index[1], jnp.full((EPAD - E,), 1 << 20, jnp.int32)])
    zrows = jnp.zeros((ROWS_PER_TILE, DP), f32)

    bvec1 = jnp.full((16,), beta1, f32)
    bvec2 = jnp.full((16,), beta2, f32)
    bscal1 = jnp.full((8, 128), beta1, f32)
    bscal2 = jnp.full((8, 128), beta2, f32)

    hp0 = _tc1(xp, W1p, b1p)
    w1 = _sc_edge_w(hp0, srcp, dstp, bvec1)
    acc1 = _sc_scatter(hp0, srcp, dstp, w1, zrows)
    hp1 = _tc2(acc1, hp0, bscal1)
    w2 = _sc_edge_w(hp1, srcp, dstp, bvec2)
    acc2 = _sc_scatter(hp1, srcp, dstp, w2, zrows)
    out = _tc3(acc2, hp1, bscal2, W2p, b2p)
    return out[:, 0:1]
